# baseline (speedup 1.0000x reference)
import functools

import jax
import jax.numpy as jnp
from jax import lax
from jax.experimental import pallas as pl
from jax.experimental.pallas import tpu as pltpu
from jax.experimental.pallas import tpu_sc as plsc

N_NODES = 10000
N_EDGES = 320000
D = 128
NC = 2
NS = 16
NHALF = N_NODES // NC
TRASH = NHALF
H_ROWS = NHALF + 8
E_PER_T = N_EDGES // NS
CHUNK = 80
SEG_E = 4000
SEGS = E_PER_T // SEG_E
CP_CAP = SEG_E + 96
ROWS_PER_TILE = 312
ZROWS = 104
TAIL_OFF = NS * ROWS_PER_TILE
TAIL_ROWS = NHALF - TAIL_OFF


def _sc_segment(x, src, dst, ew):
    mesh = plsc.VectorSubcoreMesh(core_axis_name="c", subcore_axis_name="s")

    @functools.partial(
        pl.kernel,
        out_type=jax.ShapeDtypeStruct((NC, NHALF, D), jnp.float32),
        mesh=mesh,
        compiler_params=pltpu.CompilerParams(needs_layout_passes=False),
        scratch_types=[
            [pltpu.VMEM((SEG_E,), jnp.int32) for _ in range(2)],
            [pltpu.VMEM((SEG_E,), jnp.int32) for _ in range(2)],
            [pltpu.VMEM((SEG_E,), jnp.float32) for _ in range(2)],
            pltpu.VMEM((CP_CAP,), jnp.int32),
            pltpu.VMEM((CP_CAP,), jnp.int32),
            pltpu.VMEM((CP_CAP,), jnp.float32),
            [pltpu.VMEM((CHUNK,), jnp.int32) for _ in range(2)],
            [pltpu.VMEM((CHUNK, D), jnp.float32) for _ in range(2)],
            pltpu.VMEM((ZROWS, D), jnp.float32),
            pltpu.VMEM_SHARED((H_ROWS, D), jnp.float32),
            [pltpu.SemaphoreType.DMA for _ in range(2)],
            [pltpu.SemaphoreType.DMA for _ in range(2)],
        ],
    )
    def k(x_hbm, src_hbm, dst_hbm, ew_hbm, out_hbm,
          src_s, dst_s, ew_s, src_cp, dst_cp, ew_cp, dstc, rows,
          zbuf_v, h_sh, isems, gsems):
        cid = lax.axis_index("c")
        sid = lax.axis_index("s")

        ebase = sid * E_PER_T
        row_lo = cid * NHALF

        def issue_slab(s, q):
            off = ebase + s * SEG_E
            pltpu.async_copy(src_hbm.at[pl.ds(off, SEG_E)], src_s[q], isems[q])
            pltpu.async_copy(dst_hbm.at[pl.ds(off, SEG_E)], dst_s[q], isems[q])
            pltpu.async_copy(ew_hbm.at[pl.ds(off, SEG_E)], ew_s[q], isems[q])

        def wait_slab(q):
            pltpu.make_async_copy(src_hbm.at[pl.ds(0, SEG_E)], src_s[q], isems[q]).wait()
            pltpu.make_async_copy(dst_hbm.at[pl.ds(0, SEG_E)], dst_s[q], isems[q]).wait()
            pltpu.make_async_copy(ew_hbm.at[pl.ds(0, SEG_E)], ew_s[q], isems[q]).wait()

        def issue_gather(c, p):
            pltpu.async_copy(x_hbm.at[src_cp.at[pl.ds(c * CHUNK, CHUNK)]],
                             rows[p], gsems[p])

        def wait_gather(p):
            pltpu.make_async_copy(x_hbm.at[pl.ds(0, CHUNK)], rows[p], gsems[p]).wait()

        def process(c, p):
            rb = rows[p]
            db = dstc[p]

            @plsc.parallel_loop(0, CHUNK // 16)
            def grp(g):
                off = g * 16
                base = c * CHUNK + off
                db[pl.ds(off, 16)] = dst_cp[pl.ds(base, 16)]
                w16 = ew_cp[pl.ds(base, 16)]
                for e2 in range(16):
                    e = off + e2
                    wb = jnp.full((16,), w16[e2])
                    for j in range(D // 16):
                        rb[e, pl.ds(j * 16, 16)] = rb[e, pl.ds(j * 16, 16)] * wb

        issue_slab(0, 0)
        zero16 = jnp.zeros((16,), jnp.float32)

        def zrow(r, _):
            for j in range(D // 16):
                zbuf_v[r, pl.ds(j * 16, 16)] = zero16
            return 0

        lax.fori_loop(0, ZROWS, zrow, 0)
        for kk in range(ROWS_PER_TILE // ZROWS):
            pltpu.sync_copy(zbuf_v, h_sh.at[pl.ds(sid * ROWS_PER_TILE + kk * ZROWS, ZROWS)])

        @pl.when(sid == NS - 1)
        def _zero_tail():
            pltpu.sync_copy(zbuf_v.at[pl.ds(0, TAIL_ROWS + 8)],
                            h_sh.at[pl.ds(TAIL_OFF, TAIL_ROWS + 8)])

        plsc.subcore_barrier()

        for s in range(SEGS):
            q = s % 2
            wait_slab(q)
            if s + 1 < SEGS:
                issue_slab(s + 1, 1 - q)

            sq, dq, wq = src_s[q], dst_s[q], ew_s[q]

            def crow(r, wp):
                rbase = r * CHUNK
                for g in range(CHUNK // 16):
                    off = rbase + g * 16
                    s16 = sq[pl.ds(off, 16)]
                    d16 = dq[pl.ds(off, 16)]
                    w16 = wq[pl.ds(off, 16)]
                    dl = d16 - row_lo
                    ok = (dl >= 0) & (dl < NHALF)
                    plsc.store_compressed(src_cp.at[pl.ds(wp, 16)], s16, mask=ok)
                    plsc.store_compressed(dst_cp.at[pl.ds(wp, 16)], dl, mask=ok)
                    plsc.store_compressed(ew_cp.at[pl.ds(wp, 16)], w16, mask=ok)
                    wp = wp + plsc.all_reduce_population_count(ok)[0]
                return wp

            wp = lax.fori_loop(0, SEG_E // CHUNK, crow, jnp.int32(0))

            for g in range(CHUNK // 16):
                src_cp[pl.ds(wp + g * 16, 16)] = jnp.zeros((16,), jnp.int32)
                dst_cp[pl.ds(wp + g * 16, 16)] = jnp.full((16,), TRASH, jnp.int32)
                ew_cp[pl.ds(wp + g * 16, 16)] = jnp.zeros((16,), jnp.float32)
            nch = (wp + (CHUNK - 1)) // CHUNK

            @pl.when(nch > 0)
            def _prime():
                issue_gather(0, 0)

            def chunk_body(c, _):
                par = lax.rem(c, 2)
                for p in range(2):
                    @pl.when(par == p)
                    def _slot():
                        @pl.when(c + 1 < nch)
                        def _g():
                            issue_gather(c + 1, 1 - p)
                        wait_gather(p)
                        process(c, p)
                        pltpu.sync_copy(rows[p], h_sh.at[dstc[p]], add=True)
                return 0

            lax.fori_loop(0, nch, chunk_body, 0)

        plsc.subcore_barrier()

        for kk in range(ROWS_PER_TILE // ZROWS):
            off = sid * ROWS_PER_TILE + kk * ZROWS
            pltpu.sync_copy(h_sh.at[pl.ds(off, ZROWS)], zbuf_v)
            pltpu.sync_copy(zbuf_v, out_hbm.at[cid, pl.ds(off, ZROWS)])

        @pl.when(sid == NS - 1)
        def _copy_tail():
            pltpu.sync_copy(h_sh.at[pl.ds(TAIL_OFF, TAIL_ROWS)],
                            rows[0].at[pl.ds(0, TAIL_ROWS)])
            pltpu.sync_copy(rows[0].at[pl.ds(0, TAIL_ROWS)],
                            out_hbm.at[cid, pl.ds(TAIL_OFF, TAIL_ROWS)])

    return k(x, src, dst, ew)


_TC_BLK = 1000


def _tc_linear(hpart, W, b2):
    def body(h_ref, w_ref, b_ref, o_ref):
        o_ref[...] = lax.dot_general(
            h_ref[0], w_ref[...], (((1,), (1,)), ((), ())),
            preferred_element_type=jnp.float32) + b_ref[...]

    nblk = NHALF // _TC_BLK

    return pl.pallas_call(
        body,
        grid=(N_NODES // _TC_BLK,),
        in_specs=[
            pl.BlockSpec((1, _TC_BLK, D), lambda i: (i // nblk, i % nblk, 0)),
            pl.BlockSpec((D, D), lambda i: (0, 0)),
            pl.BlockSpec((1, D), lambda i: (0, 0)),
        ],
        out_specs=pl.BlockSpec((_TC_BLK, D), lambda i: (i, 0)),
        out_shape=jax.ShapeDtypeStruct((N_NODES, D), jnp.float32),
    )(hpart, W, b2)


def kernel(x, edge_index, edge_weights, W, b):
    ei = edge_index.astype(jnp.int32)
    src = ei[0]
    dst = ei[1]
    ew = edge_weights.reshape(-1)
    hpart = _sc_segment(x, src, dst, ew)
    return _tc_linear(hpart, W, b.reshape(1, D))


# speedup vs baseline: 1.0376x; 1.0376x over previous
import functools

import jax
import jax.numpy as jnp
from jax import lax
from jax.experimental import pallas as pl
from jax.experimental.pallas import tpu as pltpu
from jax.experimental.pallas import tpu_sc as plsc

N_NODES = 10000
N_EDGES = 320000
D = 128
NC = 2
NS = 16
NHALF = N_NODES // NC
TRASH = NHALF
H_ROWS = NHALF + 8
E_PER_T = N_EDGES // NS
CHUNK = 80
SEG_E = 4000
SEGS = E_PER_T // SEG_E
CP_CAP = SEG_E + 96
ROWS_PER_TILE = 312
ZROWS = 24
TAIL_OFF = NS * ROWS_PER_TILE
TAIL_ROWS = NHALF - TAIL_OFF


def _sc_segment(x, src, dst, ew):
    mesh = plsc.VectorSubcoreMesh(core_axis_name="c", subcore_axis_name="s")

    @functools.partial(
        pl.kernel,
        out_type=jax.ShapeDtypeStruct((NC, NHALF, D), jnp.float32),
        mesh=mesh,
        compiler_params=pltpu.CompilerParams(needs_layout_passes=False),
        scratch_types=[
            [pltpu.VMEM((SEG_E,), jnp.int32) for _ in range(2)],
            [pltpu.VMEM((SEG_E,), jnp.int32) for _ in range(2)],
            [pltpu.VMEM((SEG_E,), jnp.float32) for _ in range(2)],
            pltpu.VMEM((CP_CAP,), jnp.int32),
            pltpu.VMEM((CP_CAP,), jnp.int32),
            pltpu.VMEM((CP_CAP,), jnp.float32),
            [pltpu.VMEM((CHUNK,), jnp.int32) for _ in range(4)],
            [pltpu.VMEM((CHUNK, D), jnp.float32) for _ in range(4)],
            pltpu.VMEM((ZROWS, D), jnp.float32),
            pltpu.VMEM_SHARED((H_ROWS, D), jnp.float32),
            [pltpu.SemaphoreType.DMA for _ in range(2)],
            [pltpu.SemaphoreType.DMA for _ in range(4)],
            [pltpu.SemaphoreType.DMA for _ in range(4)],
        ],
    )
    def k(x_hbm, src_hbm, dst_hbm, ew_hbm, out_hbm,
          src_s, dst_s, ew_s, src_cp, dst_cp, ew_cp, dstc, rows,
          zbuf_v, h_sh, isems, gsems, ssems):
        cid = lax.axis_index("c")
        sid = lax.axis_index("s")

        ebase = sid * E_PER_T
        row_lo = cid * NHALF

        def issue_slab(s, q):
            off = ebase + s * SEG_E
            pltpu.async_copy(src_hbm.at[pl.ds(off, SEG_E)], src_s[q], isems[q])
            pltpu.async_copy(dst_hbm.at[pl.ds(off, SEG_E)], dst_s[q], isems[q])
            pltpu.async_copy(ew_hbm.at[pl.ds(off, SEG_E)], ew_s[q], isems[q])

        def wait_slab(q):
            pltpu.make_async_copy(src_hbm.at[pl.ds(0, SEG_E)], src_s[q], isems[q]).wait()
            pltpu.make_async_copy(dst_hbm.at[pl.ds(0, SEG_E)], dst_s[q], isems[q]).wait()
            pltpu.make_async_copy(ew_hbm.at[pl.ds(0, SEG_E)], ew_s[q], isems[q]).wait()

        def issue_gather(c, p):
            pltpu.async_copy(x_hbm.at[src_cp.at[pl.ds(c * CHUNK, CHUNK)]],
                             rows[p], gsems[p])

        def wait_gather(p):
            pltpu.make_async_copy(x_hbm.at[pl.ds(0, CHUNK)], rows[p], gsems[p]).wait()

        def issue_scatter(p):
            pltpu.async_copy(rows[p], h_sh.at[dstc[p]], ssems[p], add=True)

        def wait_scatter(p):
            pltpu.make_async_copy(rows[p], h_sh.at[pl.ds(0, CHUNK)], ssems[p]).wait()

        def process(c, p):
            rb = rows[p]
            db = dstc[p]
            cbase = c * CHUNK
            for g in range(CHUNK // 16):
                db[pl.ds(g * 16, 16)] = dst_cp[pl.ds(cbase + g * 16, 16)]

            @plsc.parallel_loop(0, CHUNK, unroll=4)
            def edge(e):
                wb = plsc.load_gather(ew_cp, [jnp.full((16,), cbase + e, jnp.int32)])
                for j in range(D // 16):
                    rb[e, pl.ds(j * 16, 16)] = rb[e, pl.ds(j * 16, 16)] * wb

        issue_slab(0, 0)
        zero16 = jnp.zeros((16,), jnp.float32)

        def zrow(r, _):
            for j in range(D // 16):
                zbuf_v[r, pl.ds(j * 16, 16)] = zero16
            return 0

        lax.fori_loop(0, ZROWS, zrow, 0)
        for kk in range(ROWS_PER_TILE // ZROWS):
            pltpu.sync_copy(zbuf_v, h_sh.at[pl.ds(sid * ROWS_PER_TILE + kk * ZROWS, ZROWS)])

        @pl.when(sid == NS - 1)
        def _zero_tail():
            pltpu.sync_copy(zbuf_v.at[pl.ds(0, TAIL_ROWS + 8)],
                            h_sh.at[pl.ds(TAIL_OFF, TAIL_ROWS + 8)])

        plsc.subcore_barrier()

        for s in range(SEGS):
            q = s % 2
            wait_slab(q)
            if s + 1 < SEGS:
                issue_slab(s + 1, 1 - q)

            sq, dq, wq = src_s[q], dst_s[q], ew_s[q]

            def crow(r, wp):
                rbase = r * CHUNK
                for g in range(CHUNK // 16):
                    off = rbase + g * 16
                    s16 = sq[pl.ds(off, 16)]
                    d16 = dq[pl.ds(off, 16)]
                    w16 = wq[pl.ds(off, 16)]
                    dl = d16 - row_lo
                    ok = (dl >= 0) & (dl < NHALF)
                    plsc.store_compressed(src_cp.at[pl.ds(wp, 16)], s16, mask=ok)
                    plsc.store_compressed(dst_cp.at[pl.ds(wp, 16)], dl, mask=ok)
                    plsc.store_compressed(ew_cp.at[pl.ds(wp, 16)], w16, mask=ok)
                    wp = wp + plsc.all_reduce_population_count(ok)[0]
                return wp

            wp = lax.fori_loop(0, SEG_E // CHUNK, crow, jnp.int32(0))

            for g in range(CHUNK // 16):
                src_cp[pl.ds(wp + g * 16, 16)] = jnp.zeros((16,), jnp.int32)
                dst_cp[pl.ds(wp + g * 16, 16)] = jnp.full((16,), TRASH, jnp.int32)
                ew_cp[pl.ds(wp + g * 16, 16)] = jnp.zeros((16,), jnp.float32)
            nch = (wp + (CHUNK - 1)) // CHUNK

            @pl.when(nch > 0)
            def _prime0():
                issue_gather(0, 0)

            @pl.when(nch > 1)
            def _prime1():
                issue_gather(1, 1)

            def chunk_body(c, _):
                par = lax.rem(c, 4)
                for p in range(4):
                    g2 = (p + 2) % 4

                    @pl.when(par == p)
                    def _slot():
                        @pl.when(c + 2 < nch)
                        def _g():
                            @pl.when(c >= 2)
                            def _ws():
                                wait_scatter(g2)
                            issue_gather(c + 2, g2)
                        wait_gather(p)
                        process(c, p)
                        issue_scatter(p)
                return 0

            lax.fori_loop(0, nch, chunk_body, 0)
            for p in range(4):
                @pl.when(nch > p)
                def _drain():
                    wait_scatter(p)

        plsc.subcore_barrier()

        for kk in range(ROWS_PER_TILE // ZROWS):
            off = sid * ROWS_PER_TILE + kk * ZROWS
            pltpu.sync_copy(h_sh.at[pl.ds(off, ZROWS)], zbuf_v)
            pltpu.sync_copy(zbuf_v, out_hbm.at[cid, pl.ds(off, ZROWS)])

        @pl.when(sid == NS - 1)
        def _copy_tail():
            pltpu.sync_copy(h_sh.at[pl.ds(TAIL_OFF, TAIL_ROWS)],
                            rows[0].at[pl.ds(0, TAIL_ROWS)])
            pltpu.sync_copy(rows[0].at[pl.ds(0, TAIL_ROWS)],
                            out_hbm.at[cid, pl.ds(TAIL_OFF, TAIL_ROWS)])

    return k(x, src, dst, ew)


_TC_BLK = 1000


def _tc_linear(hpart, W, b2):
    def body(h_ref, w_ref, b_ref, o_ref):
        o_ref[...] = lax.dot_general(
            h_ref[0], w_ref[...], (((1,), (1,)), ((), ())),
            preferred_element_type=jnp.float32) + b_ref[...]

    nblk = NHALF // _TC_BLK

    return pl.pallas_call(
        body,
        grid=(N_NODES // _TC_BLK,),
        in_specs=[
            pl.BlockSpec((1, _TC_BLK, D), lambda i: (i // nblk, i % nblk, 0)),
            pl.BlockSpec((D, D), lambda i: (0, 0)),
            pl.BlockSpec((1, D), lambda i: (0, 0)),
        ],
        out_specs=pl.BlockSpec((_TC_BLK, D), lambda i: (i, 0)),
        out_shape=jax.ShapeDtypeStruct((N_NODES, D), jnp.float32),
    )(hpart, W, b2)


def kernel(x, edge_index, edge_weights, W, b):
    ei = edge_index.astype(jnp.int32)
    src = ei[0]
    dst = ei[1]
    ew = edge_weights.reshape(-1)
    hpart = _sc_segment(x, src, dst, ew)
    return _tc_linear(hpart, W, b.reshape(1, D))
